# baseline (speedup 1.0000x reference)
import jax
import jax.numpy as jnp
from jax import lax
from jax.experimental import pallas as pl
from jax.experimental.pallas import tpu as pltpu
from jax.experimental.pallas import tpu_sc as plsc

NUM_EMBEDDINGS = 1000000
EMBEDDING_DIM = 64
BATCH = 16384
N_FIELDS = 26

NC, NS = 2, 16
NW = NC * NS
ROWS_PER_W = BATCH // NW
NBUF = 2


def _gather_kernel(idx_hbm, table_hbm, out_hbm, idx_2d, idx_cols, rows_v, sems):
    wid = lax.axis_index("s") * NC + lax.axis_index("c")
    base = wid * ROWS_PER_W

    pltpu.sync_copy(idx_hbm.at[pl.ds(base, ROWS_PER_W)], idx_2d)
    lane = lax.iota(jnp.int32, 16)

    def shuffle_field(f, _):
        def shuffle_16(j, _):
            rows = j * 16 + lane
            cols = jnp.full((16,), 0, jnp.int32) + f
            v = plsc.load_gather(idx_2d, [rows, cols])
            idx_cols[f, pl.ds(j * 16, 16)] = v
            return ()

        lax.fori_loop(0, ROWS_PER_W // 16, shuffle_16, ())
        return ()

    lax.fori_loop(0, N_FIELDS, shuffle_field, ())

    for b in range(NBUF):
        pltpu.async_copy(
            table_hbm.at[idx_cols.at[b]],
            rows_v.at[b],
            sems.at[b],
        )

    def step(i, _):
        for b in range(NBUF):
            f = i + b
            pltpu.make_async_copy(
                table_hbm.at[idx_cols.at[f]],
                rows_v.at[b],
                sems.at[b],
            ).wait()
            pltpu.sync_copy(
                rows_v.at[b],
                out_hbm.at[pl.ds(base, ROWS_PER_W), f],
            )
            @pl.when(f + NBUF < N_FIELDS)
            def _():
                pltpu.async_copy(
                    table_hbm.at[idx_cols.at[f + NBUF]],
                    rows_v.at[b],
                    sems.at[b],
                )
        return ()

    lax.fori_loop(0, N_FIELDS // NBUF, lambda i, c: step(i * NBUF, c), ())


@jax.jit
def _embedding_gather(idx, weight):
    mesh = plsc.VectorSubcoreMesh(core_axis_name="c", subcore_axis_name="s")
    return pl.kernel(
        _gather_kernel,
        out_type=jax.ShapeDtypeStruct((BATCH, N_FIELDS, EMBEDDING_DIM), jnp.float32),
        mesh=mesh,
        scratch_types=[
            pltpu.VMEM((ROWS_PER_W, N_FIELDS), jnp.int32),
            pltpu.VMEM((N_FIELDS, ROWS_PER_W), jnp.int32),
            pltpu.VMEM((NBUF, ROWS_PER_W, EMBEDDING_DIM), jnp.float32),
            pltpu.SemaphoreType.DMA((NBUF,)),
        ],
        compiler_params=pltpu.CompilerParams(use_tc_tiling_on_sc=False, needs_layout_passes=False),
    )(idx, weight)


def kernel(input_, num_global_tokens, weight):
    del num_global_tokens
    return _embedding_gather(input_, weight)
